# baseline (speedup 1.0000x reference)
import functools

import jax
import jax.numpy as jnp
from jax import lax
from jax.experimental import pallas as pl
from jax.experimental.pallas import tpu as pltpu
from jax.experimental.pallas import tpu_sc as plsc

N = 10000
E = 160000
D = 256
DE = 16
TP_OUT = 384
SCAL = 128
NC = 2
NS = 16
NW = NC * NS
LANES = 16

N_PAD = 10240
WIN = N_PAD // NW
SEG = 2000
NSEG = E // SEG
CH = 16
ACC_W = WIN * D


def _pq_body(x_ref, w_ref, o_ref):
    o_ref[...] = jnp.dot(x_ref[...], w_ref[...],
                         preferred_element_type=jnp.float32)


def _edge_body(ea_ref, we_ref, w1_ref, b1_ref, w2_ref, b2_ref, w3_ref,
               b3_ref, r_ref, w_ref):
    ea = ea_ref[...]
    r_ref[...] = jnp.dot(ea, we_ref[...], preferred_element_type=jnp.float32)
    h = jax.nn.silu(jnp.dot(ea, w1_ref[...],
                            preferred_element_type=jnp.float32) + b1_ref[...])
    h = jax.nn.silu(jnp.dot(h, w2_ref[...],
                            preferred_element_type=jnp.float32) + b2_ref[...])
    w_ref[...] = jnp.dot(h, w3_ref[...],
                         preferred_element_type=jnp.float32) + b3_ref[...]


def _post_body(a_ref, w_ref, b_ref, o_ref):
    o_ref[...] = jnp.dot(a_ref[...], w_ref[...],
                         preferred_element_type=jnp.float32) + b_ref[...]


def _sc_body(src_h, dst_h, p_h, q_h, r_h, w_h, acc_h,
             srcv, dstv, sel_eid, sel_src, sel_dst,
             buf_p, buf_q, buf_r, buf_w, acc, sem):
    c = lax.axis_index("c")
    s = lax.axis_index("s")
    w = c * NS + s
    lo = w * WIN

    zvec = jnp.zeros((LANES,), jnp.float32)

    def zero_body(i, _):
        acc[pl.ds(i * LANES, LANES)] = zvec
        return 0

    lax.fori_loop(0, (ACC_W + D) // LANES, zero_body, 0)

    def seg_body(g, _):
        ebase = g * SEG
        pltpu.sync_copy(src_h.at[pl.ds(ebase, SEG)], srcv)
        pltpu.sync_copy(dst_h.at[pl.ds(ebase, SEG)], dstv)

        def sel_body(i, cnt):
            sv = srcv[pl.ds(i * LANES, LANES)]
            m = (sv >= lo) & (sv < lo + WIN)
            inc = m.astype(jnp.int32)
            nhit = jnp.sum(inc)

            @pl.when(nhit > 0)
            def _store():
                dv = dstv[pl.ds(i * LANES, LANES)]
                eid = ebase + i * LANES + lax.iota(jnp.int32, LANES)
                pos = cnt + plsc.cumsum(inc) - 1
                plsc.store_scatter(sel_eid, [pos], eid, mask=m)
                plsc.store_scatter(sel_src, [pos], sv, mask=m)
                plsc.store_scatter(sel_dst, [pos], dv, mask=m)

            return cnt + nhit

        cnt = lax.fori_loop(0, SEG // LANES, sel_body, jnp.int32(0))

        sel_eid[pl.ds(cnt, LANES)] = jnp.zeros((LANES,), jnp.int32)
        sel_src[pl.ds(cnt, LANES)] = jnp.full((LANES,), lo + WIN, jnp.int32)
        sel_dst[pl.ds(cnt, LANES)] = jnp.zeros((LANES,), jnp.int32)

        nch = (cnt + CH - 1) // CH

        def chunk_body(ci, _):
            eidv = sel_eid[pl.ds(ci * CH, CH)]
            srcg = sel_src[pl.ds(ci * CH, CH)]
            dstg = sel_dst[pl.ds(ci * CH, CH)]
            gsrc = jnp.minimum(srcg, jnp.int32(N_PAD - 1))
            cp_p = pltpu.async_copy(p_h.at[gsrc], buf_p, sem)
            cp_q = pltpu.async_copy(q_h.at[dstg], buf_q, sem)
            cp_r = pltpu.async_copy(r_h.at[eidv], buf_r, sem)
            cp_w = pltpu.async_copy(w_h.at[eidv], buf_w, sem)
            cp_p.wait()
            cp_q.wait()
            cp_r.wait()
            cp_w.wait()

            rowoff = (srcg - lo) * D
            for j in range(CH):
                base = rowoff[j]
                for k in range(SCAL // LANES):
                    sl = pl.ds(k * LANES, LANES)
                    u = buf_p[j, sl] + buf_q[j, sl] + buf_r[j, sl]
                    y = u / (1.0 + jnp.exp(-u)) * buf_w[j, sl]
                    plsc.addupdate(acc.at[pl.ds(base + k * LANES, LANES)], y)
                for k in range(SCAL // LANES):
                    gsl = pl.ds(SCAL + k * LANES, LANES)
                    dsl = pl.ds(2 * SCAL + k * LANES, LANES)
                    osl = pl.ds(SCAL + k * LANES, LANES)
                    ug = buf_p[j, gsl] + buf_q[j, gsl] + buf_r[j, gsl]
                    ud = buf_p[j, dsl] + buf_q[j, dsl] + buf_r[j, dsl]
                    y = ud * buf_w[j, osl] / (1.0 + jnp.exp(-ug))
                    plsc.addupdate(
                        acc.at[pl.ds(base + SCAL + k * LANES, LANES)], y)
            return 0

        lax.fori_loop(0, nch, chunk_body, 0)
        return 0

    lax.fori_loop(0, NSEG, seg_body, 0)

    pltpu.sync_copy(acc.at[pl.ds(0, ACC_W)], acc_h.at[pl.ds(w * ACC_W, ACC_W)])


def _make_sc_kernel():
    return functools.partial(
        pl.kernel,
        mesh=plsc.VectorSubcoreMesh(core_axis_name="c", subcore_axis_name="s"),
        out_type=jax.ShapeDtypeStruct((N_PAD * D,), jnp.float32),
        scratch_types=[
            pltpu.VMEM((SEG,), jnp.int32),
            pltpu.VMEM((SEG,), jnp.int32),
            pltpu.VMEM((SEG + 2 * LANES,), jnp.int32),
            pltpu.VMEM((SEG + 2 * LANES,), jnp.int32),
            pltpu.VMEM((SEG + 2 * LANES,), jnp.int32),
            pltpu.VMEM((CH, TP_OUT), jnp.float32),
            pltpu.VMEM((CH, TP_OUT), jnp.float32),
            pltpu.VMEM((CH, TP_OUT), jnp.float32),
            pltpu.VMEM((CH, D), jnp.float32),
            pltpu.VMEM((ACC_W + D,), jnp.float32),
            pltpu.SemaphoreType.DMA,
        ],
        compiler_params=pltpu.CompilerParams(needs_layout_passes=False),
    )(_sc_body)


def kernel(x, edge_attr, W_tp, W1, b1, W2, b2, W3, b3, W_post, b_post,
           edge_index):
    src = edge_index[0]
    dst = edge_index[1]

    x_pad = jnp.pad(x, ((0, N_PAD - N), (0, 0)))
    w_sd = jnp.concatenate([W_tp[:D], W_tp[D:2 * D]], axis=1)
    w_e = W_tp[2 * D:]

    pq = pl.pallas_call(
        _pq_body,
        grid=(N_PAD // 512,),
        in_specs=[
            pl.BlockSpec((512, D), lambda i: (i, 0)),
            pl.BlockSpec((D, 2 * TP_OUT), lambda i: (0, 0)),
        ],
        out_specs=pl.BlockSpec((512, 2 * TP_OUT), lambda i: (i, 0)),
        out_shape=jax.ShapeDtypeStruct((N_PAD, 2 * TP_OUT), jnp.float32),
    )(x_pad, w_sd)
    p = pq[:, :TP_OUT]
    q = pq[:, TP_OUT:]

    eb = 2000
    r_edges, w_edges = pl.pallas_call(
        _edge_body,
        grid=(E // eb,),
        in_specs=[
            pl.BlockSpec((eb, DE), lambda i: (i, 0)),
            pl.BlockSpec((DE, TP_OUT), lambda i: (0, 0)),
            pl.BlockSpec((DE, 64), lambda i: (0, 0)),
            pl.BlockSpec((1, 64), lambda i: (0, 0)),
            pl.BlockSpec((64, 64), lambda i: (0, 0)),
            pl.BlockSpec((1, 64), lambda i: (0, 0)),
            pl.BlockSpec((64, D), lambda i: (0, 0)),
            pl.BlockSpec((1, D), lambda i: (0, 0)),
        ],
        out_specs=[
            pl.BlockSpec((eb, TP_OUT), lambda i: (i, 0)),
            pl.BlockSpec((eb, D), lambda i: (i, 0)),
        ],
        out_shape=[
            jax.ShapeDtypeStruct((E, TP_OUT), jnp.float32),
            jax.ShapeDtypeStruct((E, D), jnp.float32),
        ],
    )(edge_attr, w_e, W1, b1.reshape(1, 64), W2, b2.reshape(1, 64),
      W3, b3.reshape(1, D))

    acc = _make_sc_kernel()(src, dst, p, q, r_edges, w_edges)
    acc = acc.reshape(N_PAD, D)

    out_pad = pl.pallas_call(
        _post_body,
        grid=(N_PAD // 512,),
        in_specs=[
            pl.BlockSpec((512, D), lambda i: (i, 0)),
            pl.BlockSpec((D, D), lambda i: (0, 0)),
            pl.BlockSpec((1, D), lambda i: (0, 0)),
        ],
        out_specs=pl.BlockSpec((512, D), lambda i: (i, 0)),
        out_shape=jax.ShapeDtypeStruct((N_PAD, D), jnp.float32),
    )(acc, W_post, b_post.reshape(1, D))

    return out_pad[:N]


# speedup vs baseline: 1.5209x; 1.5209x over previous
import functools

import jax
import jax.numpy as jnp
from jax import lax
from jax.experimental import pallas as pl
from jax.experimental.pallas import tpu as pltpu
from jax.experimental.pallas import tpu_sc as plsc

N = 10000
E = 160000
D = 256
DE = 16
TP_OUT = 384
TW = TP_OUT + D
SCAL = 128
NC = 2
NS = 16
NW = NC * NS
LANES = 16

N_PAD = 10240
WIN = N_PAD // NW
SEG = 640
NSEG = E // SEG
CH = 32
ACC_W = WIN * D


def _pq_body(x_ref, w_ref, o_ref):
    o_ref[...] = jnp.dot(x_ref[...], w_ref[...],
                         preferred_element_type=jnp.float32)


def _edge_body(ea_ref, we_ref, w1_ref, b1_ref, w2_ref, b2_ref, w3_ref,
               b3_ref, t_ref):
    ea = ea_ref[...]
    t_ref[:, :TP_OUT] = jnp.dot(ea, we_ref[...],
                                preferred_element_type=jnp.float32)
    h = jax.nn.silu(jnp.dot(ea, w1_ref[...],
                            preferred_element_type=jnp.float32) + b1_ref[...])
    h = jax.nn.silu(jnp.dot(h, w2_ref[...],
                            preferred_element_type=jnp.float32) + b2_ref[...])
    t_ref[:, TP_OUT:] = jnp.dot(h, w3_ref[...],
                                preferred_element_type=jnp.float32) + b3_ref[...]


def _post_body(a_ref, w_ref, b_ref, o_ref):
    o_ref[...] = jnp.dot(a_ref[...], w_ref[...],
                         preferred_element_type=jnp.float32) + b_ref[...]


def _sc_body(src_h, dst_h, p_h, q_h, t_h, acc_h,
             srcv, dstv, sel_eid, sel_src, sel_dst, gsrc_v,
             buf_p, buf_q, buf_t, acc, sem):
    c = lax.axis_index("c")
    s = lax.axis_index("s")
    w = c * NS + s
    lo = w * WIN

    zvec = jnp.zeros((LANES,), jnp.float32)

    def zero_body(i, _):
        acc[pl.ds(i * LANES, LANES)] = zvec
        return 0

    lax.fori_loop(0, (ACC_W + D) // LANES, zero_body, 0)

    def process_chunks(nch):
        def chunk_body(ci, _):
            for h in range(CH // LANES):
                sv = sel_src[pl.ds(ci * CH + h * LANES, LANES)]
                gsrc_v[pl.ds(h * LANES, LANES)] = jnp.minimum(
                    sv, jnp.int32(N_PAD - 1))
            cp_p = pltpu.async_copy(p_h.at[gsrc_v], buf_p, sem)
            cp_q = pltpu.async_copy(
                q_h.at[sel_dst.at[pl.ds(ci * CH, CH)]], buf_q, sem)
            cp_t = pltpu.async_copy(
                t_h.at[sel_eid.at[pl.ds(ci * CH, CH)]], buf_t, sem)
            cp_p.wait()
            cp_q.wait()
            cp_t.wait()

            def edge_body(j, _):
                rv = sel_src[pl.ds(ci * CH + j, LANES)]
                base = (rv[0] - lo) * D
                for k in range(SCAL // LANES):
                    sl = pl.ds(k * LANES, LANES)
                    u = buf_p[j, sl] + buf_q[j, sl] + buf_t[j, sl]
                    y = u / (1.0 + jnp.exp(-u)) \
                        * buf_t[j, pl.ds(TP_OUT + k * LANES, LANES)]
                    plsc.addupdate(
                        acc.at[pl.ds(base + k * LANES, LANES)], y)
                for k in range(SCAL // LANES):
                    gsl = pl.ds(SCAL + k * LANES, LANES)
                    dsl = pl.ds(2 * SCAL + k * LANES, LANES)
                    ug = buf_p[j, gsl] + buf_q[j, gsl] + buf_t[j, gsl]
                    ud = buf_p[j, dsl] + buf_q[j, dsl] + buf_t[j, dsl]
                    y = ud * buf_t[j, pl.ds(TP_OUT + SCAL + k * LANES,
                                            LANES)] \
                        / (1.0 + jnp.exp(-ug))
                    plsc.addupdate(
                        acc.at[pl.ds(base + SCAL + k * LANES, LANES)], y)
                return 0

            lax.fori_loop(0, CH, edge_body, 0)
            return 0

        lax.fori_loop(0, nch, chunk_body, 0)

    def seg_body(g, cnt):
        ebase = g * SEG
        pltpu.sync_copy(src_h.at[pl.ds(ebase, SEG)], srcv)
        pltpu.sync_copy(dst_h.at[pl.ds(ebase, SEG)], dstv)

        def sel_body(i, cc):
            sv = srcv[pl.ds(i * LANES, LANES)]
            m = (sv >= lo) & (sv < lo + WIN)
            inc = m.astype(jnp.int32)
            nhit = jnp.sum(inc)

            @pl.when(nhit > 0)
            def _store():
                dv = dstv[pl.ds(i * LANES, LANES)]
                eid = ebase + i * LANES + lax.iota(jnp.int32, LANES)
                pos = cc + plsc.cumsum(inc) - 1
                plsc.store_scatter(sel_eid, [pos], eid, mask=m)
                plsc.store_scatter(sel_src, [pos], sv, mask=m)
                plsc.store_scatter(sel_dst, [pos], dv, mask=m)

            return cc + nhit

        cnt = lax.fori_loop(0, SEG // LANES, sel_body, cnt)

        nfull = cnt // CH
        process_chunks(nfull)

        rem = cnt - nfull * CH

        @pl.when(nfull > 0)
        def _move():
            for h in range(CH // LANES):
                ev = sel_eid[pl.ds(nfull * CH + h * LANES, LANES)]
                sv = sel_src[pl.ds(nfull * CH + h * LANES, LANES)]
                dv = sel_dst[pl.ds(nfull * CH + h * LANES, LANES)]
                sel_eid[pl.ds(h * LANES, LANES)] = ev
                sel_src[pl.ds(h * LANES, LANES)] = sv
                sel_dst[pl.ds(h * LANES, LANES)] = dv

        return rem

    cnt = lax.fori_loop(0, NSEG, seg_body, jnp.int32(0))

    zpad = jnp.zeros((LANES,), jnp.int32)
    tpad = jnp.full((LANES,), lo + WIN, jnp.int32)
    for h in range(CH // LANES):
        sel_eid[pl.ds(cnt + h * LANES, LANES)] = zpad
        sel_src[pl.ds(cnt + h * LANES, LANES)] = tpad
        sel_dst[pl.ds(cnt + h * LANES, LANES)] = zpad
    process_chunks((cnt + CH - 1) // CH)

    pltpu.sync_copy(acc.at[pl.ds(0, ACC_W)], acc_h.at[pl.ds(w * ACC_W, ACC_W)])


def _make_sc_kernel():
    return functools.partial(
        pl.kernel,
        mesh=plsc.VectorSubcoreMesh(core_axis_name="c", subcore_axis_name="s"),
        out_type=jax.ShapeDtypeStruct((N_PAD * D,), jnp.float32),
        scratch_types=[
            pltpu.VMEM((SEG,), jnp.int32),
            pltpu.VMEM((SEG,), jnp.int32),
            pltpu.VMEM((SEG + 2 * CH,), jnp.int32),
            pltpu.VMEM((SEG + 2 * CH,), jnp.int32),
            pltpu.VMEM((SEG + 2 * CH,), jnp.int32),
            pltpu.VMEM((CH,), jnp.int32),
            pltpu.VMEM((CH, TP_OUT), jnp.float32),
            pltpu.VMEM((CH, TP_OUT), jnp.float32),
            pltpu.VMEM((CH, TW), jnp.float32),
            pltpu.VMEM((ACC_W + D,), jnp.float32),
            pltpu.SemaphoreType.DMA,
        ],
        compiler_params=pltpu.CompilerParams(needs_layout_passes=False),
    )(_sc_body)


def kernel(x, edge_attr, W_tp, W1, b1, W2, b2, W3, b3, W_post, b_post,
           edge_index):
    src = edge_index[0]
    dst = edge_index[1]

    x_pad = jnp.pad(x, ((0, N_PAD - N), (0, 0)))
    w_sd = jnp.concatenate([W_tp[:D], W_tp[D:2 * D]], axis=1)
    w_e = W_tp[2 * D:]

    pq = pl.pallas_call(
        _pq_body,
        grid=(N_PAD // 512,),
        in_specs=[
            pl.BlockSpec((512, D), lambda i: (i, 0)),
            pl.BlockSpec((D, 2 * TP_OUT), lambda i: (0, 0)),
        ],
        out_specs=pl.BlockSpec((512, 2 * TP_OUT), lambda i: (i, 0)),
        out_shape=jax.ShapeDtypeStruct((N_PAD, 2 * TP_OUT), jnp.float32),
    )(x_pad, w_sd)
    p = pq[:, :TP_OUT]
    q = pq[:, TP_OUT:]

    eb = 2000
    t_edges = pl.pallas_call(
        _edge_body,
        grid=(E // eb,),
        in_specs=[
            pl.BlockSpec((eb, DE), lambda i: (i, 0)),
            pl.BlockSpec((DE, TP_OUT), lambda i: (0, 0)),
            pl.BlockSpec((DE, 64), lambda i: (0, 0)),
            pl.BlockSpec((1, 64), lambda i: (0, 0)),
            pl.BlockSpec((64, 64), lambda i: (0, 0)),
            pl.BlockSpec((1, 64), lambda i: (0, 0)),
            pl.BlockSpec((64, D), lambda i: (0, 0)),
            pl.BlockSpec((1, D), lambda i: (0, 0)),
        ],
        out_specs=pl.BlockSpec((eb, TW), lambda i: (i, 0)),
        out_shape=jax.ShapeDtypeStruct((E, TW), jnp.float32),
    )(edge_attr, w_e, W1, b1.reshape(1, 64), W2, b2.reshape(1, 64),
      W3, b3.reshape(1, D))

    acc = _make_sc_kernel()(src, dst, p, q, t_edges)
    acc = acc.reshape(N_PAD, D)

    out_pad = pl.pallas_call(
        _post_body,
        grid=(N_PAD // 512,),
        in_specs=[
            pl.BlockSpec((512, D), lambda i: (i, 0)),
            pl.BlockSpec((D, D), lambda i: (0, 0)),
            pl.BlockSpec((1, D), lambda i: (0, 0)),
        ],
        out_specs=pl.BlockSpec((512, D), lambda i: (i, 0)),
        out_shape=jax.ShapeDtypeStruct((N_PAD, D), jnp.float32),
    )(acc, W_post, b_post.reshape(1, D))

    return out_pad[:N]
